# baseline (speedup 1.0000x reference)
import functools

import jax
import jax.numpy as jnp
from jax import lax
from jax.experimental import pallas as pl
from jax.experimental.pallas import tpu as pltpu
from jax.experimental.pallas import tpu_sc as plsc

B = 128
LT = 64
LI = 197
D = 768
NC = 2
NS = 16
NW = NC * NS
RPW = B // NW
L = 16
BIG = 1 << 30


def _row_argmax(simbuf, j):
    lanes = lax.broadcasted_iota(jnp.int32, (L,), 0)
    best_val = jnp.full((L,), -jnp.inf, dtype=jnp.float32)
    best_idx = jnp.zeros((L,), dtype=jnp.int32)
    for k in range(B // L):
        v = simbuf[j, pl.ds(k * L, L)]
        idxs = lanes + (k * L)
        better = v > best_val
        best_val = jnp.where(better, v, best_val)
        best_idx = jnp.where(better, idxs, best_idx)
    m = jnp.max(best_val)
    cand = jnp.where(best_val == m, best_idx, BIG)
    return jnp.min(cand)


def _sc_body(text_hbm, tattn_hbm, img_hbm, iattn_hbm, t2i_hbm, i2t_hbm,
             out_text, out_tattn, out_img, out_iattn,
             sim_t2i_v, sim_i2t_v, gsem):
    c = lax.axis_index("c")
    s = lax.axis_index("s")
    wid = c * NS + s
    base = wid * RPW

    pltpu.sync_copy(t2i_hbm.at[pl.ds(base, RPW)], sim_t2i_v)
    pltpu.sync_copy(i2t_hbm.at[pl.ds(base, RPW)], sim_i2t_v)

    img_idx = [_row_argmax(sim_t2i_v, j) for j in range(RPW)]
    txt_idx = [_row_argmax(sim_i2t_v, j) for j in range(RPW)]

    copies = []
    for j in range(RPW):
        r = base + j
        ii = img_idx[j]
        ti = txt_idx[j]
        copies.append(pltpu.async_copy(img_hbm.at[r], out_img.at[r], gsem))
        copies.append(pltpu.async_copy(text_hbm.at[r], out_text.at[r], gsem))
        copies.append(pltpu.async_copy(iattn_hbm.at[r], out_iattn.at[r], gsem))
        copies.append(pltpu.async_copy(tattn_hbm.at[r], out_tattn.at[r], gsem))
        copies.append(pltpu.async_copy(img_hbm.at[ii], out_img.at[B + r], gsem))
        copies.append(pltpu.async_copy(text_hbm.at[ti], out_text.at[B + r], gsem))
        copies.append(pltpu.async_copy(iattn_hbm.at[ii], out_iattn.at[B + r], gsem))
        copies.append(pltpu.async_copy(tattn_hbm.at[ti], out_tattn.at[B + r], gsem))
    for cp in copies:
        cp.wait()


@jax.jit
def kernel(text_embeddings, text_attns, image_embeddings, image_attns,
           sim_t2i, sim_i2t):
    mesh = plsc.VectorSubcoreMesh(
        core_axis_name="c", subcore_axis_name="s",
        num_cores=NC, num_subcores=NS)
    out_type = (
        jax.ShapeDtypeStruct((2 * B, LT, D), jnp.float32),
        jax.ShapeDtypeStruct((2 * B, LT), jnp.float32),
        jax.ShapeDtypeStruct((2 * B, LI, D), jnp.float32),
        jax.ShapeDtypeStruct((2 * B, LI), jnp.float32),
    )
    scratch = [
        pltpu.VMEM((RPW, B), jnp.float32),
        pltpu.VMEM((RPW, B), jnp.float32),
        pltpu.SemaphoreType.DMA,
    ]
    run = pl.kernel(
        _sc_body, out_type=out_type, mesh=mesh, scratch_types=scratch,
        compiler_params=pltpu.CompilerParams(needs_layout_passes=False),
        name="neg_data_collector_sc")
    return run(text_embeddings, text_attns, image_embeddings, image_attns,
               sim_t2i, sim_i2t)


# speedup vs baseline: 19.1894x; 19.1894x over previous
import functools

import jax
import jax.numpy as jnp
from jax import lax
from jax.experimental import pallas as pl
from jax.experimental.pallas import tpu as pltpu
from jax.experimental.pallas import tpu_sc as plsc

B = 128
LT = 64
LI = 197
D = 768
NC = 2
NS = 16
NW = NC * NS
RPW = B // NW
L = 16
BIG = 1 << 30


def _row_argmax(simbuf, j):
    lanes = lax.broadcasted_iota(jnp.int32, (L,), 0)
    best_val = jnp.full((L,), -jnp.inf, dtype=jnp.float32)
    best_idx = jnp.zeros((L,), dtype=jnp.int32)
    for k in range(B // L):
        v = simbuf[j, pl.ds(k * L, L)]
        idxs = lanes + (k * L)
        better = v > best_val
        best_val = jnp.where(better, v, best_val)
        best_idx = jnp.where(better, idxs, best_idx)
    m = jnp.max(best_val)
    cand = jnp.where(best_val == m, best_idx, BIG)
    return jnp.min(cand)


CHUNK = 64


def _sc_body(text_hbm, tattn_hbm, img_hbm, iattn_hbm, t2i_hbm, i2t_hbm,
             out_text, out_tattn, out_img, out_iattn,
             sim_t2i_v, sim_i2t_v, buf0, buf1, gsem,
             isem0, isem1, osem0, osem1):
    c = lax.axis_index("c")
    s = lax.axis_index("s")
    wid = c * NS + s
    base = wid * RPW

    pltpu.sync_copy(t2i_hbm.at[pl.ds(base, RPW)], sim_t2i_v)
    pltpu.sync_copy(i2t_hbm.at[pl.ds(base, RPW)], sim_i2t_v)

    img_idx = [_row_argmax(sim_t2i_v, j) for j in range(RPW)]
    txt_idx = [_row_argmax(sim_i2t_v, j) for j in range(RPW)]

    copies = []
    for j in range(RPW):
        r = base + j
        copies.append(pltpu.async_copy(iattn_hbm.at[r], out_iattn.at[r], gsem))
        copies.append(pltpu.async_copy(tattn_hbm.at[r], out_tattn.at[r], gsem))
        copies.append(
            pltpu.async_copy(iattn_hbm.at[img_idx[j]], out_iattn.at[B + r], gsem))
        copies.append(
            pltpu.async_copy(tattn_hbm.at[txt_idx[j]], out_tattn.at[B + r], gsem))

    chunks = []
    for j in range(RPW):
        r = base + j
        for (tbl, out, srow, drow, lseq) in (
                (img_hbm, out_img, r, r, LI),
                (img_hbm, out_img, img_idx[j], B + r, LI),
                (text_hbm, out_text, r, r, LT),
                (text_hbm, out_text, txt_idx[j], B + r, LT)):
            o = 0
            while o < lseq:
                n = min(CHUNK, lseq - o)
                chunks.append((tbl.at[srow, pl.ds(o, n)],
                               out.at[drow, pl.ds(o, n)], n))
                o += n

    bufs = (buf0, buf1)
    isems = (isem0, isem1)
    osems = (osem0, osem1)
    pending = [None, None]
    for i, (src, dst, n) in enumerate(chunks):
        b = i % 2
        if pending[b] is not None:
            pending[b].wait()
        pltpu.async_copy(src, bufs[b].at[pl.ds(0, n)], isems[b]).wait()
        pending[b] = pltpu.async_copy(bufs[b].at[pl.ds(0, n)], dst, osems[b])
    for h in pending:
        h.wait()
    for cp in copies:
        cp.wait()


@jax.jit
def kernel(text_embeddings, text_attns, image_embeddings, image_attns,
           sim_t2i, sim_i2t):
    mesh = plsc.VectorSubcoreMesh(
        core_axis_name="c", subcore_axis_name="s",
        num_cores=NC, num_subcores=NS)
    out_type = (
        jax.ShapeDtypeStruct((2 * B, LT, D), jnp.float32),
        jax.ShapeDtypeStruct((2 * B, LT), jnp.float32),
        jax.ShapeDtypeStruct((2 * B, LI, D), jnp.float32),
        jax.ShapeDtypeStruct((2 * B, LI), jnp.float32),
    )
    scratch = [
        pltpu.VMEM((RPW, B), jnp.float32),
        pltpu.VMEM((RPW, B), jnp.float32),
        pltpu.VMEM((CHUNK, D), jnp.float32),
        pltpu.VMEM((CHUNK, D), jnp.float32),
        pltpu.SemaphoreType.DMA,
        pltpu.SemaphoreType.DMA,
        pltpu.SemaphoreType.DMA,
        pltpu.SemaphoreType.DMA,
        pltpu.SemaphoreType.DMA,
    ]
    run = pl.kernel(
        _sc_body, out_type=out_type, mesh=mesh, scratch_types=scratch,
        compiler_params=pltpu.CompilerParams(needs_layout_passes=False),
        name="neg_data_collector_sc")
    return run(text_embeddings, text_attns, image_embeddings, image_attns,
               sim_t2i, sim_i2t)
